# baseline (speedup 1.0000x reference)
import jax
import jax.numpy as jnp
from jax import lax
from jax.experimental import pallas as pl
from jax.experimental.pallas import tpu as pltpu
from jax.experimental.pallas import tpu_sc as plsc

N_NODES = 50000
N_EDGES = 800000
D_IN = 128
D_MSG = 64
D_HALF = D_MSG // 2

NUM_SUBCORES = 16
N_PAD = 50176
ROWS_PER_TEC = N_PAD // NUM_SUBCORES
N_PW = N_PAD // 32
SCH = N_PW // 2
N_TRASH = N_PAD - N_NODES

CHUNK = 128
SB = 2
CW = 8
SB_E = SB * CHUNK
E_PT = 50176
NSB = E_PT // SB_E
NSB_C = NSB // 2

ROW_BLK = 2000


def _mm_body(h_ref, w_ref, b_ref, o0_ref, o1_ref):
    x = jnp.dot(h_ref[...], w_ref[...], preferred_element_type=jnp.float32)
    x = x + b_ref[...]
    s = jnp.maximum(x, 0.0) + jnp.log1p(jnp.exp(-jnp.abs(x)))
    o0_ref[...] = s[:, :D_HALF]
    o1_ref[...] = s[:, D_HALF:]


def _matmul_softplus(h, wt, b):
    grid = (N_NODES // ROW_BLK,)
    return pl.pallas_call(
        _mm_body,
        grid=grid,
        in_specs=[
            pl.BlockSpec((ROW_BLK, D_IN), lambda i: (i, 0)),
            pl.BlockSpec((D_IN, D_MSG), lambda i: (0, 0)),
            pl.BlockSpec((1, D_MSG), lambda i: (0, 0)),
        ],
        out_specs=[
            pl.BlockSpec((ROW_BLK, D_HALF), lambda i: (i, 0)),
            pl.BlockSpec((ROW_BLK, D_HALF), lambda i: (i, 0)),
        ],
        out_shape=[
            jax.ShapeDtypeStruct((N_NODES, D_HALF), jnp.float32),
            jax.ShapeDtypeStruct((N_NODES, D_HALF), jnp.float32),
        ],
    )(h, wt, b)


def _sc_sum_body(src_hbm, dst_hbm, t0_hbm, t1_hbm, z32_hbm,
                 sum0_hbm, sum1_hbm,
                 sidx, didx, rows, acc, gsem, ssem, isem):
    c = lax.axis_index("c")
    s = lax.axis_index("s")

    base_r = s * ROWS_PER_TEC
    pltpu.sync_copy(z32_hbm, acc.at[pl.ds(base_r, ROWS_PER_TEC)])
    plsc.subcore_barrier()

    sb0 = s * NSB

    def fire_gathers(j, b):
        for k in range(SB):
            @pl.when(c == 0)
            def _(k=k):
                pltpu.async_copy(t0_hbm.at[sidx.at[j, k]], rows.at[b, k],
                                 gsem)

            @pl.when(c == 1)
            def _(k=k):
                pltpu.async_copy(t1_hbm.at[sidx.at[j, k]], rows.at[b, k],
                                 gsem)

    def drain_gathers(j, b):
        for k in range(SB):
            pltpu.make_async_copy(t0_hbm.at[sidx.at[j, k]], rows.at[b, k],
                                  gsem).wait()

    def drain_scatters(j, b):
        for k in range(SB):
            pltpu.make_async_copy(rows.at[b, k], acc.at[didx.at[j, k]],
                                  ssem).wait()

    def load_idx(i, j):
        pltpu.async_copy(src_hbm.at[i], sidx.at[j], isem)
        pltpu.async_copy(dst_hbm.at[i], didx.at[j], isem)

    def wait_idx(i, j):
        pltpu.make_async_copy(src_hbm.at[i], sidx.at[j], isem).wait()
        pltpu.make_async_copy(dst_hbm.at[i], didx.at[j], isem).wait()

    pltpu.sync_copy(src_hbm.at[sb0], sidx.at[0])
    pltpu.sync_copy(dst_hbm.at[sb0], didx.at[0])
    load_idx(sb0 + 1, 1)
    fire_gathers(0, 0)
    wait_idx(sb0 + 1, 1)
    load_idx(sb0 + 2, 2)
    fire_gathers(1, 1)

    def body(sb, carry):
        b = sb % 3
        bm1 = (sb + 2) % 3
        j = sb % 4
        jm1 = (sb + 3) % 4
        jp2 = (sb + 2) % 4
        jp3 = (sb + 3) % 4
        ip2 = sb0 + jnp.minimum(sb + 2, NSB - 1)
        ip3 = sb0 + jnp.minimum(sb + 3, NSB - 1)

        @pl.when(sb > 0)
        def _():
            drain_scatters(jm1, bm1)

        drain_gathers(j, b)

        for k in range(SB):
            pltpu.async_copy(rows.at[b, k], acc.at[didx.at[j, k]],
                             ssem, add=True)

        wait_idx(ip2, jp2)
        fire_gathers(jp2, bm1)
        load_idx(ip3, jp3)
        return carry

    lax.fori_loop(0, NSB, body, 0)

    drain_scatters((NSB + 3) % 4, (NSB + 2) % 3)
    drain_gathers(NSB % 4, NSB % 3)
    drain_gathers((NSB + 1) % 4, (NSB + 1) % 3)
    last = sb0 + NSB - 1
    wait_idx(last, (NSB + 2) % 4)

    plsc.subcore_barrier()

    row_slice = pl.ds(base_r, ROWS_PER_TEC)

    @pl.when(c == 0)
    def _():
        pltpu.sync_copy(acc.at[row_slice], sum0_hbm.at[row_slice])

    @pl.when(c == 1)
    def _():
        pltpu.sync_copy(acc.at[row_slice], sum1_hbm.at[row_slice])


def _segment_sums(src, dst, t0, t1):
    z32 = jnp.zeros((ROWS_PER_TEC, D_HALF), jnp.float32)
    mesh = plsc.VectorSubcoreMesh(core_axis_name="c", subcore_axis_name="s")
    f = pl.kernel(
        _sc_sum_body,
        out_type=[
            jax.ShapeDtypeStruct((N_PAD, D_HALF), jnp.float32),
            jax.ShapeDtypeStruct((N_PAD, D_HALF), jnp.float32),
        ],
        mesh=mesh,
        compiler_params=pltpu.CompilerParams(use_tc_tiling_on_sc=False),
        scratch_types=[
            pltpu.VMEM((4, SB, CHUNK), jnp.int32),
            pltpu.VMEM((4, SB, CHUNK), jnp.int32),
            pltpu.VMEM((3, SB, CHUNK, D_HALF), jnp.float32),
            pltpu.VMEM_SHARED((N_PAD, D_HALF), jnp.float32),
            pltpu.SemaphoreType.DMA,
            pltpu.SemaphoreType.DMA,
            pltpu.SemaphoreType.DMA,
        ],
    )
    return f(src, dst, t0, t1, z32)


def _sc_cnt_body(dst_hbm, z8_hbm, ones_hbm, cnt_hbm,
                 didx, ones, cnt, ssem, isem):
    c = lax.axis_index("c")
    s = lax.axis_index("s")

    base_r = s * ROWS_PER_TEC
    pltpu.sync_copy(z8_hbm, cnt.at[pl.ds(base_r, ROWS_PER_TEC)])
    pltpu.sync_copy(ones_hbm, ones)
    plsc.subcore_barrier()

    sb0 = s * NSB + c * NSB_C

    pltpu.sync_copy(dst_hbm.at[sb0], didx.at[0])
    pltpu.async_copy(dst_hbm.at[sb0 + 1], didx.at[1], isem)
    pltpu.async_copy(dst_hbm.at[sb0 + 2], didx.at[2], isem)

    def body(sb, carry):
        j = sb % 5
        jm2 = (sb + 3) % 5
        jp3 = (sb + 3) % 5
        i3 = sb0 + jnp.minimum(sb + 3, NSB_C - 1)

        @pl.when(sb > 1)
        def _():
            for k in range(SB):
                pltpu.make_async_copy(ones, cnt.at[didx.at[jm2, k]],
                                      ssem).wait()

        for k in range(SB):
            pltpu.async_copy(ones, cnt.at[didx.at[j, k]], ssem, add=True)

        pltpu.make_async_copy(dst_hbm.at[i3], didx.at[jp3], isem).wait()
        pltpu.async_copy(dst_hbm.at[i3], didx.at[jp3], isem)
        return carry

    lax.fori_loop(0, NSB_C, body, 0)

    for sbt in (NSB_C - 2, NSB_C - 1):
        for k in range(SB):
            pltpu.make_async_copy(ones, cnt.at[didx.at[sbt % 5, k]],
                                  ssem).wait()
    last = sb0 + NSB_C - 1
    pltpu.make_async_copy(dst_hbm.at[last], didx.at[(NSB_C + 2) % 5],
                          isem).wait()
    pltpu.make_async_copy(dst_hbm.at[last], didx.at[(NSB_C + 3) % 5],
                          isem).wait()

    plsc.subcore_barrier()

    row_slice = pl.ds(base_r, ROWS_PER_TEC)
    @pl.when(c == 0)
    def _():
        pltpu.sync_copy(cnt.at[row_slice], cnt_hbm.at[0, row_slice])

    @pl.when(c == 1)
    def _():
        pltpu.sync_copy(cnt.at[row_slice], cnt_hbm.at[1, row_slice])


def _segment_counts(dst):
    z8 = jnp.zeros((ROWS_PER_TEC, CW), jnp.float32)
    ones = jnp.ones((CHUNK, CW), jnp.float32)
    mesh = plsc.VectorSubcoreMesh(core_axis_name="c", subcore_axis_name="s")
    f = pl.kernel(
        _sc_cnt_body,
        out_type=jax.ShapeDtypeStruct((2, N_PAD, CW), jnp.float32),
        mesh=mesh,
        compiler_params=pltpu.CompilerParams(use_tc_tiling_on_sc=False),
        scratch_types=[
            pltpu.VMEM((5, SB, CHUNK), jnp.int32),
            pltpu.VMEM((CHUNK, CW), jnp.float32),
            pltpu.VMEM_SHARED((N_PAD, CW), jnp.float32),
            pltpu.SemaphoreType.DMA,
            pltpu.SemaphoreType.DMA,
        ],
    )
    return f(dst, z8, ones)


def _sc_scale_body(s0_hbm, s1_hbm, cnt_hbm, out_hbm,
                   a0, a1, cb0, cb1, ibuf, obuf, dsem):
    c = lax.axis_index("c")
    s = lax.axis_index("s")
    wid = s * 2 + c

    def chunk(ci):
        base = wid * N_PW + ci * SCH
        sl = pl.ds(base, SCH)
        pltpu.sync_copy(s0_hbm.at[sl], a0)
        pltpu.sync_copy(s1_hbm.at[sl], a1)
        pltpu.sync_copy(cnt_hbm.at[0, sl], cb0)
        pltpu.sync_copy(cnt_hbm.at[1, sl], cb1)

        zeros16 = jnp.zeros((16,), jnp.int32)

        def group(g, carry):
            ridx = lax.iota(jnp.int32, 16) + g * 16
            c0 = plsc.load_gather(cb0, [ridx, zeros16])
            c1 = plsc.load_gather(cb1, [ridx, zeros16])
            inv = 1.0 / jnp.maximum(c0 + c1, 1.0)
            plsc.store_scatter(ibuf, [ridx], inv)
            return carry

        lax.fori_loop(0, SCH // 16, group, 0)

        def row(r, carry):
            iv = plsc.load_gather(ibuf, [jnp.full((16,), r, jnp.int32)])
            obuf[r, 0:16] = a0[r, 0:16] * iv
            obuf[r, 16:32] = a0[r, 16:32] * iv
            obuf[r, 32:48] = a1[r, 0:16] * iv
            obuf[r, 48:64] = a1[r, 16:32] * iv
            return carry

        lax.fori_loop(0, SCH, row, 0)
        pltpu.sync_copy(obuf, out_hbm.at[sl])

    chunk(0)
    chunk(1)


def _scale_interleave(sum0, sum1, cnt2):
    mesh = plsc.VectorSubcoreMesh(core_axis_name="c", subcore_axis_name="s")
    f = pl.kernel(
        _sc_scale_body,
        out_type=jax.ShapeDtypeStruct((N_PAD, D_MSG), jnp.float32),
        mesh=mesh,
        compiler_params=pltpu.CompilerParams(use_tc_tiling_on_sc=False,
                                             needs_layout_passes=False),
        scratch_types=[
            pltpu.VMEM((SCH, D_HALF), jnp.float32),
            pltpu.VMEM((SCH, D_HALF), jnp.float32),
            pltpu.VMEM((SCH, CW), jnp.float32),
            pltpu.VMEM((SCH, CW), jnp.float32),
            pltpu.VMEM((SCH,), jnp.float32),
            pltpu.VMEM((SCH, D_MSG), jnp.float32),
            pltpu.SemaphoreType.DMA,
        ],
    )
    return f(sum0, sum1, cnt2)


def _pad_edges(src, dst):
    per_tec = N_EDGES // NUM_SUBCORES
    pad_n = E_PT - per_tec
    trash = N_NODES + (jnp.arange(pad_n, dtype=jnp.int32) % N_TRASH)
    src2 = jnp.concatenate(
        [src.reshape(NUM_SUBCORES, per_tec),
         jnp.zeros((NUM_SUBCORES, pad_n), jnp.int32)], axis=1)
    dst2 = jnp.concatenate(
        [dst.reshape(NUM_SUBCORES, per_tec),
         jnp.broadcast_to(trash, (NUM_SUBCORES, pad_n))], axis=1)
    return (src2.reshape(NUM_SUBCORES * NSB, SB, CHUNK),
            dst2.reshape(NUM_SUBCORES * NSB, SB, CHUNK))


@jax.jit
def kernel(h, edge_index, edge_weight, edge_attr, data, W0, b0, Ws, bs):
    src = edge_index[0]
    dst = edge_index[1]
    src2, dst2 = _pad_edges(src, dst)
    cnt2 = _segment_counts(dst2)
    cnt2, h = lax.optimization_barrier((cnt2, h))
    t0, t1 = _matmul_softplus(h, W0.T, b0.reshape(1, D_MSG))
    sum0, sum1 = _segment_sums(src2, dst2, t0, t1)
    return _scale_interleave(sum0, sum1, cnt2)[:N_NODES]


# speedup vs baseline: 1.1118x; 1.1118x over previous
import jax
import jax.numpy as jnp
from jax import lax
from jax.experimental import pallas as pl
from jax.experimental.pallas import tpu as pltpu
from jax.experimental.pallas import tpu_sc as plsc

N_NODES = 50000
N_EDGES = 800000
D_IN = 128
D_MSG = 64
D_HALF = D_MSG // 2

NUM_SUBCORES = 16
N_PAD = 50176
ROWS_PER_TEC = N_PAD // NUM_SUBCORES
N_PW = N_PAD // 32
SCH = N_PW // 2
N_TRASH = N_PAD - N_NODES

CHUNK = 128
SB = 2
CW = 8
SB_E = SB * CHUNK
E_PT = 50176
NSB = E_PT // SB_E
NSB_C = NSB // 2

ROW_BLK = 2000


def _mm_body(h_ref, w_ref, b_ref, o0_ref, o1_ref):
    x = jnp.dot(h_ref[...], w_ref[...], preferred_element_type=jnp.float32)
    x = x + b_ref[...]
    s = jnp.maximum(x, 0.0) + jnp.log1p(jnp.exp(-jnp.abs(x)))
    o0_ref[...] = s[:, :D_HALF]
    o1_ref[...] = s[:, D_HALF:]


def _matmul_softplus(h, wt, b):
    grid = (N_NODES // ROW_BLK,)
    return pl.pallas_call(
        _mm_body,
        grid=grid,
        in_specs=[
            pl.BlockSpec((ROW_BLK, D_IN), lambda i: (i, 0)),
            pl.BlockSpec((D_IN, D_MSG), lambda i: (0, 0)),
            pl.BlockSpec((1, D_MSG), lambda i: (0, 0)),
        ],
        out_specs=[
            pl.BlockSpec((ROW_BLK, D_HALF), lambda i: (i, 0)),
            pl.BlockSpec((ROW_BLK, D_HALF), lambda i: (i, 0)),
        ],
        out_shape=[
            jax.ShapeDtypeStruct((N_NODES, D_HALF), jnp.float32),
            jax.ShapeDtypeStruct((N_NODES, D_HALF), jnp.float32),
        ],
    )(h, wt, b)


def _sc_sum_body(src_hbm, dst_hbm, t0_hbm, t1_hbm, z32_hbm,
                 sum0_hbm, sum1_hbm,
                 sidx, didx, rows, acc, gsem, ssem, isem):
    c = lax.axis_index("c")
    s = lax.axis_index("s")

    base_r = s * ROWS_PER_TEC
    pltpu.sync_copy(z32_hbm, acc.at[pl.ds(base_r, ROWS_PER_TEC)])
    plsc.subcore_barrier()

    sb0 = s * NSB

    def fire_gathers(j, b):
        for k in range(SB):
            @pl.when(c == 0)
            def _(k=k):
                pltpu.async_copy(t0_hbm.at[sidx.at[j, k]], rows.at[b, k],
                                 gsem)

            @pl.when(c == 1)
            def _(k=k):
                pltpu.async_copy(t1_hbm.at[sidx.at[j, k]], rows.at[b, k],
                                 gsem)

    def drain_gathers(j, b):
        for k in range(SB):
            pltpu.make_async_copy(t0_hbm.at[sidx.at[j, k]], rows.at[b, k],
                                  gsem).wait()

    def drain_scatters(j, b):
        for k in range(SB):
            pltpu.make_async_copy(rows.at[b, k], acc.at[didx.at[j, k]],
                                  ssem).wait()

    def load_idx(i, j):
        pltpu.async_copy(src_hbm.at[i], sidx.at[j], isem)
        pltpu.async_copy(dst_hbm.at[i], didx.at[j], isem)

    def wait_idx(i, j):
        pltpu.make_async_copy(src_hbm.at[i], sidx.at[j], isem).wait()
        pltpu.make_async_copy(dst_hbm.at[i], didx.at[j], isem).wait()

    pltpu.sync_copy(src_hbm.at[sb0], sidx.at[0])
    pltpu.sync_copy(dst_hbm.at[sb0], didx.at[0])
    load_idx(sb0 + 1, 1)
    fire_gathers(0, 0)
    wait_idx(sb0 + 1, 1)
    load_idx(sb0 + 2, 2)
    fire_gathers(1, 1)

    def body(sb, carry):
        b = sb % 3
        bm1 = (sb + 2) % 3
        j = sb % 4
        jm1 = (sb + 3) % 4
        jp2 = (sb + 2) % 4
        jp3 = (sb + 3) % 4
        ip2 = sb0 + jnp.minimum(sb + 2, NSB - 1)
        ip3 = sb0 + jnp.minimum(sb + 3, NSB - 1)

        @pl.when(sb > 0)
        def _():
            drain_scatters(jm1, bm1)

        drain_gathers(j, b)

        for k in range(SB):
            pltpu.async_copy(rows.at[b, k], acc.at[didx.at[j, k]],
                             ssem, add=True)

        wait_idx(ip2, jp2)
        fire_gathers(jp2, bm1)
        load_idx(ip3, jp3)
        return carry

    lax.fori_loop(0, NSB, body, 0)

    drain_scatters((NSB + 3) % 4, (NSB + 2) % 3)
    drain_gathers(NSB % 4, NSB % 3)
    drain_gathers((NSB + 1) % 4, (NSB + 1) % 3)
    last = sb0 + NSB - 1
    wait_idx(last, (NSB + 2) % 4)

    plsc.subcore_barrier()

    row_slice = pl.ds(base_r, ROWS_PER_TEC)

    @pl.when(c == 0)
    def _():
        pltpu.sync_copy(acc.at[row_slice], sum0_hbm.at[row_slice])

    @pl.when(c == 1)
    def _():
        pltpu.sync_copy(acc.at[row_slice], sum1_hbm.at[row_slice])


def _segment_sums(src, dst, t0, t1):
    z32 = jnp.zeros((ROWS_PER_TEC, D_HALF), jnp.float32)
    mesh = plsc.VectorSubcoreMesh(core_axis_name="c", subcore_axis_name="s")
    f = pl.kernel(
        _sc_sum_body,
        out_type=[
            jax.ShapeDtypeStruct((N_PAD, D_HALF), jnp.float32),
            jax.ShapeDtypeStruct((N_PAD, D_HALF), jnp.float32),
        ],
        mesh=mesh,
        compiler_params=pltpu.CompilerParams(use_tc_tiling_on_sc=False),
        scratch_types=[
            pltpu.VMEM((4, SB, CHUNK), jnp.int32),
            pltpu.VMEM((4, SB, CHUNK), jnp.int32),
            pltpu.VMEM((3, SB, CHUNK, D_HALF), jnp.float32),
            pltpu.VMEM_SHARED((N_PAD, D_HALF), jnp.float32),
            pltpu.SemaphoreType.DMA,
            pltpu.SemaphoreType.DMA,
            pltpu.SemaphoreType.DMA,
        ],
    )
    return f(src, dst, t0, t1, z32)


def _sc_cnt_body(dst_hbm, z8_hbm, ones_hbm, cnt_hbm,
                 didx, ones, cnt, ssem, isem):
    c = lax.axis_index("c")
    s = lax.axis_index("s")

    base_r = s * ROWS_PER_TEC
    pltpu.sync_copy(z8_hbm, cnt.at[pl.ds(base_r, ROWS_PER_TEC)])
    pltpu.sync_copy(ones_hbm, ones)
    plsc.subcore_barrier()

    sb0 = s * NSB + c * NSB_C

    pltpu.sync_copy(dst_hbm.at[sb0], didx.at[0])
    pltpu.async_copy(dst_hbm.at[sb0 + 1], didx.at[1], isem)
    pltpu.async_copy(dst_hbm.at[sb0 + 2], didx.at[2], isem)

    def body(sb, carry):
        j = sb % 5
        jm2 = (sb + 3) % 5
        jp3 = (sb + 3) % 5
        i3 = sb0 + jnp.minimum(sb + 3, NSB_C - 1)

        @pl.when(sb > 1)
        def _():
            for k in range(SB):
                pltpu.make_async_copy(ones, cnt.at[didx.at[jm2, k]],
                                      ssem).wait()

        for k in range(SB):
            pltpu.async_copy(ones, cnt.at[didx.at[j, k]], ssem, add=True)

        pltpu.make_async_copy(dst_hbm.at[i3], didx.at[jp3], isem).wait()
        pltpu.async_copy(dst_hbm.at[i3], didx.at[jp3], isem)
        return carry

    lax.fori_loop(0, NSB_C, body, 0)

    for sbt in (NSB_C - 2, NSB_C - 1):
        for k in range(SB):
            pltpu.make_async_copy(ones, cnt.at[didx.at[sbt % 5, k]],
                                  ssem).wait()
    last = sb0 + NSB_C - 1
    pltpu.make_async_copy(dst_hbm.at[last], didx.at[(NSB_C + 2) % 5],
                          isem).wait()
    pltpu.make_async_copy(dst_hbm.at[last], didx.at[(NSB_C + 3) % 5],
                          isem).wait()

    plsc.subcore_barrier()

    row_slice = pl.ds(base_r, ROWS_PER_TEC)
    @pl.when(c == 0)
    def _():
        pltpu.sync_copy(cnt.at[row_slice], cnt_hbm.at[0, row_slice])

    @pl.when(c == 1)
    def _():
        pltpu.sync_copy(cnt.at[row_slice], cnt_hbm.at[1, row_slice])


def _segment_counts(dst):
    z8 = jnp.zeros((ROWS_PER_TEC, CW), jnp.float32)
    ones = jnp.ones((CHUNK, CW), jnp.float32)
    mesh = plsc.VectorSubcoreMesh(core_axis_name="c", subcore_axis_name="s")
    f = pl.kernel(
        _sc_cnt_body,
        out_type=jax.ShapeDtypeStruct((2, N_PAD, CW), jnp.float32),
        mesh=mesh,
        compiler_params=pltpu.CompilerParams(use_tc_tiling_on_sc=False),
        scratch_types=[
            pltpu.VMEM((5, SB, CHUNK), jnp.int32),
            pltpu.VMEM((CHUNK, CW), jnp.float32),
            pltpu.VMEM_SHARED((N_PAD, CW), jnp.float32),
            pltpu.SemaphoreType.DMA,
            pltpu.SemaphoreType.DMA,
        ],
    )
    return f(dst, z8, ones)


def _sc_scale_body(s0_hbm, s1_hbm, cnt_hbm, out_hbm,
                   a0, a1, cb0, cb1, ibuf, obuf, dsem):
    c = lax.axis_index("c")
    s = lax.axis_index("s")
    wid = s * 2 + c

    def chunk(ci, size):
        base = wid * N_PW + ci * SCH
        sl = pl.ds(base, size)
        bsl = pl.ds(0, size)
        pltpu.sync_copy(s0_hbm.at[sl], a0.at[bsl])
        pltpu.sync_copy(s1_hbm.at[sl], a1.at[bsl])
        pltpu.sync_copy(cnt_hbm.at[0, sl], cb0.at[bsl])
        pltpu.sync_copy(cnt_hbm.at[1, sl], cb1.at[bsl])

        zeros16 = jnp.zeros((16,), jnp.int32)

        def group(g, carry):
            ridx = lax.iota(jnp.int32, 16) + g * 16
            c0 = plsc.load_gather(cb0, [ridx, zeros16])
            c1 = plsc.load_gather(cb1, [ridx, zeros16])
            inv = 1.0 / jnp.maximum(c0 + c1, 1.0)
            plsc.store_scatter(ibuf, [ridx], inv)
            return carry

        lax.fori_loop(0, size // 16, group, 0)

        def row(r, carry):
            iv = plsc.load_gather(ibuf, [jnp.full((16,), r, jnp.int32)])
            obuf[r, 0:16] = a0[r, 0:16] * iv
            obuf[r, 16:32] = a0[r, 16:32] * iv
            obuf[r, 32:48] = a1[r, 0:16] * iv
            obuf[r, 48:64] = a1[r, 16:32] * iv
            return carry

        lax.fori_loop(0, size, row, 0)
        pltpu.sync_copy(obuf.at[bsl], out_hbm.at[sl])

    chunk(0, SCH)

    @pl.when(wid < 31)
    def _():
        chunk(1, SCH)

    @pl.when(wid == 31)
    def _():
        chunk(1, N_NODES - 31 * N_PW - SCH)


def _scale_interleave(sum0, sum1, cnt2):
    mesh = plsc.VectorSubcoreMesh(core_axis_name="c", subcore_axis_name="s")
    f = pl.kernel(
        _sc_scale_body,
        out_type=jax.ShapeDtypeStruct((N_NODES, D_MSG), jnp.float32),
        mesh=mesh,
        compiler_params=pltpu.CompilerParams(use_tc_tiling_on_sc=False,
                                             needs_layout_passes=False),
        scratch_types=[
            pltpu.VMEM((SCH, D_HALF), jnp.float32),
            pltpu.VMEM((SCH, D_HALF), jnp.float32),
            pltpu.VMEM((SCH, CW), jnp.float32),
            pltpu.VMEM((SCH, CW), jnp.float32),
            pltpu.VMEM((SCH,), jnp.float32),
            pltpu.VMEM((SCH, D_MSG), jnp.float32),
            pltpu.SemaphoreType.DMA,
        ],
    )
    return f(sum0, sum1, cnt2)


def _pad_edges(src, dst):
    per_tec = N_EDGES // NUM_SUBCORES
    pad_n = E_PT - per_tec
    trash = N_NODES + (jnp.arange(pad_n, dtype=jnp.int32) % N_TRASH)
    src2 = jnp.concatenate(
        [src.reshape(NUM_SUBCORES, per_tec),
         jnp.zeros((NUM_SUBCORES, pad_n), jnp.int32)], axis=1)
    dst2 = jnp.concatenate(
        [dst.reshape(NUM_SUBCORES, per_tec),
         jnp.broadcast_to(trash, (NUM_SUBCORES, pad_n))], axis=1)
    return (src2.reshape(NUM_SUBCORES * NSB, SB, CHUNK),
            dst2.reshape(NUM_SUBCORES * NSB, SB, CHUNK))


@jax.jit
def kernel(h, edge_index, edge_weight, edge_attr, data, W0, b0, Ws, bs):
    src = edge_index[0]
    dst = edge_index[1]
    src2, dst2 = _pad_edges(src, dst)
    cnt2 = _segment_counts(dst2)
    t0, t1 = _matmul_softplus(h, W0.T, b0.reshape(1, D_MSG))
    sum0, sum1 = _segment_sums(src2, dst2, t0, t1)
    return _scale_interleave(sum0, sum1, cnt2)


# speedup vs baseline: 1.1450x; 1.0298x over previous
import jax
import jax.numpy as jnp
from jax import lax
from jax.experimental import pallas as pl
from jax.experimental.pallas import tpu as pltpu
from jax.experimental.pallas import tpu_sc as plsc

N_NODES = 50000
N_EDGES = 800000
D_IN = 128
D_MSG = 64
D_HALF = D_MSG // 2

NUM_SUBCORES = 16
N_PAD = 50048
ROWS_PER_TEC = N_PAD // NUM_SUBCORES
N_TRASH = N_PAD - N_NODES

CHUNK = 128
SB = 2
CW = 8
SB_E = SB * CHUNK
E_PT = 50176
NSB = E_PT // SB_E
NSB_C = NSB // 2

ROW_BLK = 2000


def _mm_body(h_ref, w_ref, b_ref, o0_ref, o1_ref):
    x = jnp.dot(h_ref[...], w_ref[...], preferred_element_type=jnp.float32)
    x = x + b_ref[...]
    s = jnp.maximum(x, 0.0) + jnp.log1p(jnp.exp(-jnp.abs(x)))
    o0_ref[...] = s[:, :D_HALF]
    o1_ref[...] = s[:, D_HALF:]


def _matmul_softplus(h, wt, b):
    grid = (N_NODES // ROW_BLK,)
    return pl.pallas_call(
        _mm_body,
        grid=grid,
        in_specs=[
            pl.BlockSpec((ROW_BLK, D_IN), lambda i: (i, 0)),
            pl.BlockSpec((D_IN, D_MSG), lambda i: (0, 0)),
            pl.BlockSpec((1, D_MSG), lambda i: (0, 0)),
        ],
        out_specs=[
            pl.BlockSpec((ROW_BLK, D_HALF), lambda i: (i, 0)),
            pl.BlockSpec((ROW_BLK, D_HALF), lambda i: (i, 0)),
        ],
        out_shape=[
            jax.ShapeDtypeStruct((N_NODES, D_HALF), jnp.float32),
            jax.ShapeDtypeStruct((N_NODES, D_HALF), jnp.float32),
        ],
    )(h, wt, b)


def _sc_sum_body(src_hbm, dst_hbm, t0_hbm, t1_hbm, z32_hbm,
                 sum0_hbm, sum1_hbm,
                 sidx, didx, rows, acc, gsem, ssem, isem):
    c = lax.axis_index("c")
    s = lax.axis_index("s")

    base_r = s * ROWS_PER_TEC
    pltpu.sync_copy(z32_hbm, acc.at[pl.ds(base_r, ROWS_PER_TEC)])
    plsc.subcore_barrier()

    sb0 = s * NSB

    def fire_gathers(j, b):
        for k in range(SB):
            @pl.when(c == 0)
            def _(k=k):
                pltpu.async_copy(t0_hbm.at[sidx.at[j, k]], rows.at[b, k],
                                 gsem)

            @pl.when(c == 1)
            def _(k=k):
                pltpu.async_copy(t1_hbm.at[sidx.at[j, k]], rows.at[b, k],
                                 gsem)

    def drain_gathers(j, b):
        for k in range(SB):
            pltpu.make_async_copy(t0_hbm.at[sidx.at[j, k]], rows.at[b, k],
                                  gsem).wait()

    def drain_scatters(j, b):
        for k in range(SB):
            pltpu.make_async_copy(rows.at[b, k], acc.at[didx.at[j, k]],
                                  ssem).wait()

    def load_idx(i, j):
        pltpu.async_copy(src_hbm.at[i], sidx.at[j], isem)
        pltpu.async_copy(dst_hbm.at[i], didx.at[j], isem)

    def wait_idx(i, j):
        pltpu.make_async_copy(src_hbm.at[i], sidx.at[j], isem).wait()
        pltpu.make_async_copy(dst_hbm.at[i], didx.at[j], isem).wait()

    pltpu.sync_copy(src_hbm.at[sb0], sidx.at[0])
    pltpu.sync_copy(dst_hbm.at[sb0], didx.at[0])
    load_idx(sb0 + 1, 1)
    fire_gathers(0, 0)
    wait_idx(sb0 + 1, 1)
    load_idx(sb0 + 2, 2)
    fire_gathers(1, 1)

    def body(sb, carry):
        b = sb % 3
        bm1 = (sb + 2) % 3
        j = sb % 4
        jm1 = (sb + 3) % 4
        jp2 = (sb + 2) % 4
        jp3 = (sb + 3) % 4
        ip2 = sb0 + jnp.minimum(sb + 2, NSB - 1)
        ip3 = sb0 + jnp.minimum(sb + 3, NSB - 1)

        @pl.when(sb > 0)
        def _():
            drain_scatters(jm1, bm1)

        drain_gathers(j, b)

        for k in range(SB):
            pltpu.async_copy(rows.at[b, k], acc.at[didx.at[j, k]],
                             ssem, add=True)

        wait_idx(ip2, jp2)
        fire_gathers(jp2, bm1)
        load_idx(ip3, jp3)
        return carry

    lax.fori_loop(0, NSB, body, 0)

    drain_scatters((NSB + 3) % 4, (NSB + 2) % 3)
    drain_gathers(NSB % 4, NSB % 3)
    drain_gathers((NSB + 1) % 4, (NSB + 1) % 3)
    last = sb0 + NSB - 1
    wait_idx(last, (NSB + 2) % 4)

    plsc.subcore_barrier()

    row_slice = pl.ds(base_r, ROWS_PER_TEC)

    @pl.when(c == 0)
    def _():
        pltpu.sync_copy(acc.at[row_slice], sum0_hbm.at[row_slice])

    @pl.when(c == 1)
    def _():
        pltpu.sync_copy(acc.at[row_slice], sum1_hbm.at[row_slice])


def _segment_sums(src, dst, t0, t1):
    z32 = jnp.zeros((ROWS_PER_TEC, D_HALF), jnp.float32)
    mesh = plsc.VectorSubcoreMesh(core_axis_name="c", subcore_axis_name="s")
    f = pl.kernel(
        _sc_sum_body,
        out_type=[
            jax.ShapeDtypeStruct((N_PAD, D_HALF), jnp.float32),
            jax.ShapeDtypeStruct((N_PAD, D_HALF), jnp.float32),
        ],
        mesh=mesh,
        compiler_params=pltpu.CompilerParams(use_tc_tiling_on_sc=False),
        scratch_types=[
            pltpu.VMEM((4, SB, CHUNK), jnp.int32),
            pltpu.VMEM((4, SB, CHUNK), jnp.int32),
            pltpu.VMEM((3, SB, CHUNK, D_HALF), jnp.float32),
            pltpu.VMEM_SHARED((N_PAD, D_HALF), jnp.float32),
            pltpu.SemaphoreType.DMA,
            pltpu.SemaphoreType.DMA,
            pltpu.SemaphoreType.DMA,
        ],
    )
    return f(src, dst, t0, t1, z32)


def _sc_cnt_body(dst_hbm, z8_hbm, ones_hbm, cnt_hbm,
                 didx, ones, cnt, ssem, isem):
    c = lax.axis_index("c")
    s = lax.axis_index("s")

    base_r = s * ROWS_PER_TEC
    pltpu.sync_copy(z8_hbm, cnt.at[pl.ds(base_r, ROWS_PER_TEC)])
    pltpu.sync_copy(ones_hbm, ones)
    plsc.subcore_barrier()

    sb0 = s * NSB + c * NSB_C

    pltpu.sync_copy(dst_hbm.at[sb0], didx.at[0])
    pltpu.async_copy(dst_hbm.at[sb0 + 1], didx.at[1], isem)
    pltpu.async_copy(dst_hbm.at[sb0 + 2], didx.at[2], isem)

    def body(sb, carry):
        j = sb % 5
        jm2 = (sb + 3) % 5
        jp3 = (sb + 3) % 5
        i3 = sb0 + jnp.minimum(sb + 3, NSB_C - 1)

        @pl.when(sb > 1)
        def _():
            for k in range(SB):
                pltpu.make_async_copy(ones, cnt.at[didx.at[jm2, k]],
                                      ssem).wait()

        for k in range(SB):
            pltpu.async_copy(ones, cnt.at[didx.at[j, k]], ssem, add=True)

        pltpu.make_async_copy(dst_hbm.at[i3], didx.at[jp3], isem).wait()
        pltpu.async_copy(dst_hbm.at[i3], didx.at[jp3], isem)
        return carry

    lax.fori_loop(0, NSB_C, body, 0)

    for sbt in (NSB_C - 2, NSB_C - 1):
        for k in range(SB):
            pltpu.make_async_copy(ones, cnt.at[didx.at[sbt % 5, k]],
                                  ssem).wait()
    last = sb0 + NSB_C - 1
    pltpu.make_async_copy(dst_hbm.at[last], didx.at[(NSB_C + 2) % 5],
                          isem).wait()
    pltpu.make_async_copy(dst_hbm.at[last], didx.at[(NSB_C + 3) % 5],
                          isem).wait()

    plsc.subcore_barrier()

    row_slice = pl.ds(base_r, ROWS_PER_TEC)
    @pl.when(c == 0)
    def _():
        pltpu.sync_copy(cnt.at[row_slice], cnt_hbm.at[0, row_slice])

    @pl.when(c == 1)
    def _():
        pltpu.sync_copy(cnt.at[row_slice], cnt_hbm.at[1, row_slice])


def _segment_counts(dst):
    z8 = jnp.zeros((ROWS_PER_TEC, CW), jnp.float32)
    ones = jnp.ones((CHUNK, CW), jnp.float32)
    mesh = plsc.VectorSubcoreMesh(core_axis_name="c", subcore_axis_name="s")
    f = pl.kernel(
        _sc_cnt_body,
        out_type=jax.ShapeDtypeStruct((2, N_PAD, CW), jnp.float32),
        mesh=mesh,
        compiler_params=pltpu.CompilerParams(use_tc_tiling_on_sc=False),
        scratch_types=[
            pltpu.VMEM((5, SB, CHUNK), jnp.int32),
            pltpu.VMEM((CHUNK, CW), jnp.float32),
            pltpu.VMEM_SHARED((N_PAD, CW), jnp.float32),
            pltpu.SemaphoreType.DMA,
            pltpu.SemaphoreType.DMA,
        ],
    )
    return f(dst, z8, ones)


def _combine_body(s0_ref, s1_ref, c0_ref, c1_ref, o_ref):
    cnt = c0_ref[0, :, 0:1] + c1_ref[0, :, 0:1]
    inv = 1.0 / jnp.maximum(cnt, 1.0)
    o_ref[...] = jnp.concatenate([s0_ref[...] * inv, s1_ref[...] * inv],
                                 axis=1)


def _combine(sum0, sum1, cnt2):
    grid = (N_NODES // ROW_BLK,)
    return pl.pallas_call(
        _combine_body,
        grid=grid,
        in_specs=[
            pl.BlockSpec((ROW_BLK, D_HALF), lambda i: (i, 0)),
            pl.BlockSpec((ROW_BLK, D_HALF), lambda i: (i, 0)),
            pl.BlockSpec((1, ROW_BLK, CW), lambda i: (0, i, 0)),
            pl.BlockSpec((1, ROW_BLK, CW), lambda i: (1, i, 0)),
        ],
        out_specs=pl.BlockSpec((ROW_BLK, D_MSG), lambda i: (i, 0)),
        out_shape=jax.ShapeDtypeStruct((N_NODES, D_MSG), jnp.float32),
    )(sum0, sum1, cnt2, cnt2)


def _pad_edges(src, dst):
    per_tec = N_EDGES // NUM_SUBCORES
    pad_n = E_PT - per_tec
    trash = N_NODES + (jnp.arange(pad_n, dtype=jnp.int32) % N_TRASH)
    src2 = jnp.concatenate(
        [src.reshape(NUM_SUBCORES, per_tec),
         jnp.zeros((NUM_SUBCORES, pad_n), jnp.int32)], axis=1)
    dst2 = jnp.concatenate(
        [dst.reshape(NUM_SUBCORES, per_tec),
         jnp.broadcast_to(trash, (NUM_SUBCORES, pad_n))], axis=1)
    return (src2.reshape(NUM_SUBCORES * NSB, SB, CHUNK),
            dst2.reshape(NUM_SUBCORES * NSB, SB, CHUNK))


@jax.jit
def kernel(h, edge_index, edge_weight, edge_attr, data, W0, b0, Ws, bs):
    src = edge_index[0]
    dst = edge_index[1]
    src2, dst2 = _pad_edges(src, dst)
    cnt2 = _segment_counts(dst2)
    t0, t1 = _matmul_softplus(h, W0.T, b0.reshape(1, D_MSG))
    sum0, sum1 = _segment_sums(src2, dst2, t0, t1)
    return _combine(sum0, sum1, cnt2)
